# baseline (speedup 1.0000x reference)
import functools

import jax
import jax.numpy as jnp
from jax import lax
from jax.experimental import pallas as pl
from jax.experimental.pallas import tpu as pltpu
from jax.experimental.pallas import tpu_sc as plsc

VOCAB = 1000000
SEQ_LEN = 200
DIM = 64
BATCH = 4096

NUM_CORES = 2
NUM_SUBCORES = 16
NUM_WORKERS = NUM_CORES * NUM_SUBCORES
ROWS_PER_WORKER = BATCH // NUM_WORKERS
HALF = SEQ_LEN // 2


@jax.jit
def _sc_embed(inputs2, token_table, position_table):
  mesh = plsc.VectorSubcoreMesh(
      core_axis_name="c", subcore_axis_name="s",
      num_cores=NUM_CORES, num_subcores=NUM_SUBCORES)

  @functools.partial(
      pl.kernel,
      out_type=jax.ShapeDtypeStruct((BATCH * SEQ_LEN, DIM), jnp.float32),
      mesh=mesh,
      scratch_types=[
          pltpu.VMEM((2, HALF), jnp.int32),
          pltpu.VMEM((SEQ_LEN, DIM), jnp.float32),
          pltpu.SemaphoreType.DMA,
      ],
      compiler_params=pltpu.CompilerParams(use_tc_tiling_on_sc=False),
  )
  def k(inputs_hbm, table_hbm, pos_hbm, out_hbm, idx_v, acc_v, sem):
    wid = lax.axis_index("s") * NUM_CORES + lax.axis_index("c")
    base = wid * ROWS_PER_WORKER

    @pl.loop(0, ROWS_PER_WORKER)
    def _(r):
      row = base + r
      pltpu.sync_copy(inputs_hbm.at[pl.ds(row * 2, 2)], idx_v)
      pltpu.sync_copy(pos_hbm, acc_v)
      g0 = pltpu.async_copy(
          table_hbm.at[idx_v.at[0]], acc_v.at[pl.ds(0, HALF)], sem, add=True)
      g1 = pltpu.async_copy(
          table_hbm.at[idx_v.at[1]], acc_v.at[pl.ds(HALF, HALF)], sem,
          add=True)
      g0.wait()
      g1.wait()
      pltpu.sync_copy(acc_v, out_hbm.at[pl.ds(row * SEQ_LEN, SEQ_LEN)])

  return k(inputs2, token_table, position_table)


def kernel(inputs, token_table, position_table):
  inputs2 = inputs.reshape(BATCH * 2, HALF).astype(jnp.int32)
  out = _sc_embed(inputs2, token_table, position_table)
  return out.reshape(BATCH, SEQ_LEN, DIM)


# speedup vs baseline: 1.2891x; 1.2891x over previous
import functools

import jax
import jax.numpy as jnp
from jax import lax
from jax.experimental import pallas as pl
from jax.experimental.pallas import tpu as pltpu
from jax.experimental.pallas import tpu_sc as plsc

VOCAB = 1000000
SEQ_LEN = 200
DIM = 64
BATCH = 4096

NUM_CORES = 2
NUM_SUBCORES = 16
NUM_WORKERS = NUM_CORES * NUM_SUBCORES
ROWS_PER_WORKER = BATCH // NUM_WORKERS
HALF = SEQ_LEN // 2
NBUF = 4
NGROUPS = ROWS_PER_WORKER // NBUF
LANES = 16
VPR = DIM // LANES


@jax.jit
def _sc_embed(inputs2, token_table, position_table):
  mesh = plsc.VectorSubcoreMesh(
      core_axis_name="c", subcore_axis_name="s",
      num_cores=NUM_CORES, num_subcores=NUM_SUBCORES)

  scratch = (
      [pltpu.VMEM((SEQ_LEN, DIM), jnp.float32)]
      + [pltpu.VMEM((2, HALF), jnp.int32) for _ in range(NBUF)]
      + [pltpu.VMEM((SEQ_LEN, DIM), jnp.float32) for _ in range(NBUF)]
      + [pltpu.SemaphoreType.DMA for _ in range(3 * NBUF)]
  )

  @functools.partial(
      pl.kernel,
      out_type=jax.ShapeDtypeStruct((BATCH * SEQ_LEN, DIM), jnp.float32),
      mesh=mesh,
      scratch_types=scratch,
      compiler_params=pltpu.CompilerParams(use_tc_tiling_on_sc=False),
  )
  def k(inputs_hbm, table_hbm, pos_hbm, out_hbm, pos_v, *bufs):
    idx = bufs[:NBUF]
    acc = bufs[NBUF:2 * NBUF]
    si = bufs[2 * NBUF:3 * NBUF]
    sg = bufs[3 * NBUF:4 * NBUF]
    sw = bufs[4 * NBUF:5 * NBUF]

    wid = lax.axis_index("s") * NUM_CORES + lax.axis_index("c")
    base = wid * ROWS_PER_WORKER

    pltpu.sync_copy(pos_hbm, pos_v)

    def start_idx(i, b):
      pltpu.async_copy(inputs_hbm.at[pl.ds((base + i) * 2, 2)], idx[b], si[b])

    def wait_idx(b):
      pltpu.make_async_copy(inputs_hbm.at[pl.ds(0, 2)], idx[b], si[b]).wait()

    def start_gathers(b):
      g0 = pltpu.async_copy(
          table_hbm.at[idx[b].at[0]], acc[b].at[pl.ds(0, HALF)], sg[b])
      g1 = pltpu.async_copy(
          table_hbm.at[idx[b].at[1]], acc[b].at[pl.ds(HALF, HALF)], sg[b])
      return g0, g1

    def add_pos(b):
      a = acc[b]

      @pl.loop(0, SEQ_LEN, unroll=2)
      def _(r):
        for c in range(VPR):
          sl = pl.ds(c * LANES, LANES)
          plsc.addupdate(a.at[r, sl], pos_v[r, sl])

    def start_write(i, b):
      pltpu.async_copy(
          acc[b], out_hbm.at[pl.ds((base + i) * SEQ_LEN, SEQ_LEN)], sw[b])

    def wait_write(b):
      pltpu.make_async_copy(
          acc[b], out_hbm.at[pl.ds(0, SEQ_LEN)], sw[b]).wait()

    for b in range(NBUF):
      start_idx(b, b)

    handles = []
    for b in range(NBUF):
      wait_idx(b)
      handles.append(start_gathers(b))
    for b in range(NBUF):
      handles[b][0].wait()
      handles[b][1].wait()
      add_pos(b)
      start_write(b, b)
      start_idx(NBUF + b, b)

    @pl.loop(1, NGROUPS)
    def _(o):
      i0 = o * NBUF
      hs = []
      for b in range(NBUF):
        wait_idx(b)
        wait_write(b)
        hs.append(start_gathers(b))
      for b in range(NBUF):
        hs[b][0].wait()
        hs[b][1].wait()
        add_pos(b)
        start_write(i0 + b, b)

        @pl.when(i0 + b + NBUF < ROWS_PER_WORKER)
        def _():
          start_idx(i0 + b + NBUF, b)

    for b in range(NBUF):
      wait_write(b)

  return k(inputs2, token_table, position_table)


def kernel(inputs, token_table, position_table):
  inputs2 = inputs.reshape(BATCH * 2, HALF).astype(jnp.int32)
  out = _sc_embed(inputs2, token_table, position_table)
  return out.reshape(BATCH, SEQ_LEN, DIM)


# speedup vs baseline: 1.2914x; 1.0018x over previous
import functools

import jax
import jax.numpy as jnp
from jax import lax
from jax.experimental import pallas as pl
from jax.experimental.pallas import tpu as pltpu
from jax.experimental.pallas import tpu_sc as plsc

VOCAB = 1000000
SEQ_LEN = 200
DIM = 64
BATCH = 4096

NUM_CORES = 2
NUM_SUBCORES = 16
NUM_WORKERS = NUM_CORES * NUM_SUBCORES
ROWS_PER_WORKER = BATCH // NUM_WORKERS
SPLIT = 128
REST = SEQ_LEN - SPLIT
NBUF = 4
NGROUPS = ROWS_PER_WORKER // NBUF
LANES = 16
VPR = DIM // LANES


@jax.jit
def _sc_embed(inputs, token_table, position_table):
  mesh = plsc.VectorSubcoreMesh(
      core_axis_name="c", subcore_axis_name="s",
      num_cores=NUM_CORES, num_subcores=NUM_SUBCORES)

  scratch = (
      [pltpu.VMEM((SEQ_LEN, DIM), jnp.float32)]
      + [pltpu.VMEM((SEQ_LEN,), jnp.int32) for _ in range(NBUF)]
      + [pltpu.VMEM((SEQ_LEN, DIM), jnp.float32) for _ in range(NBUF)]
      + [pltpu.SemaphoreType.DMA for _ in range(3 * NBUF)]
  )

  @functools.partial(
      pl.kernel,
      out_type=jax.ShapeDtypeStruct((BATCH, SEQ_LEN, DIM), jnp.float32),
      mesh=mesh,
      scratch_types=scratch,
      compiler_params=pltpu.CompilerParams(use_tc_tiling_on_sc=False),
  )
  def k(inputs_hbm, table_hbm, pos_hbm, out_hbm, pos_v, *bufs):
    idx = bufs[:NBUF]
    acc = bufs[NBUF:2 * NBUF]
    si = bufs[2 * NBUF:3 * NBUF]
    sg = bufs[3 * NBUF:4 * NBUF]
    sw = bufs[4 * NBUF:5 * NBUF]

    wid = lax.axis_index("s") * NUM_CORES + lax.axis_index("c")
    base = wid * ROWS_PER_WORKER

    pltpu.sync_copy(pos_hbm, pos_v)

    def start_idx(i, b):
      pltpu.async_copy(inputs_hbm.at[base + i], idx[b], si[b])

    def wait_idx(b):
      pltpu.make_async_copy(inputs_hbm.at[0], idx[b], si[b]).wait()

    def start_gathers(b):
      g0 = pltpu.async_copy(
          table_hbm.at[idx[b].at[pl.ds(0, SPLIT)]],
          acc[b].at[pl.ds(0, SPLIT)], sg[b])
      g1 = pltpu.async_copy(
          table_hbm.at[idx[b].at[pl.ds(SPLIT, REST)]],
          acc[b].at[pl.ds(SPLIT, REST)], sg[b])
      return g0, g1

    def add_pos(b):
      a = acc[b]

      @pl.loop(0, SEQ_LEN, unroll=2)
      def _(r):
        for c in range(VPR):
          sl = pl.ds(c * LANES, LANES)
          plsc.addupdate(a.at[r, sl], pos_v[r, sl])

    def start_write(i, b):
      pltpu.async_copy(acc[b], out_hbm.at[base + i], sw[b])

    def wait_write(b):
      pltpu.make_async_copy(acc[b], out_hbm.at[0], sw[b]).wait()

    for b in range(NBUF):
      start_idx(b, b)

    handles = []
    for b in range(NBUF):
      wait_idx(b)
      handles.append(start_gathers(b))
    for b in range(NBUF):
      handles[b][0].wait()
      handles[b][1].wait()
      add_pos(b)
      start_write(b, b)
      start_idx(NBUF + b, b)

    @pl.loop(1, NGROUPS)
    def _(o):
      i0 = o * NBUF
      hs = []
      for b in range(NBUF):
        wait_idx(b)
        wait_write(b)
        hs.append(start_gathers(b))
      for b in range(NBUF):
        hs[b][0].wait()
        hs[b][1].wait()
        add_pos(b)
        start_write(i0 + b, b)

        @pl.when(i0 + b + NBUF < ROWS_PER_WORKER)
        def _():
          start_idx(i0 + b + NBUF, b)

    for b in range(NBUF):
      wait_write(b)

  return k(inputs, token_table, position_table)


def kernel(inputs, token_table, position_table):
  return _sc_embed(inputs.astype(jnp.int32), token_table, position_table)


# speedup vs baseline: 1.6944x; 1.3121x over previous
import functools

import jax
import jax.numpy as jnp
from jax import lax
from jax.experimental import pallas as pl
from jax.experimental.pallas import tpu as pltpu
from jax.experimental.pallas import tpu_sc as plsc

VOCAB = 1000000
SEQ_LEN = 200
DIM = 64
BATCH = 4096

NUM_CORES = 2
NUM_SUBCORES = 16
NUM_WORKERS = NUM_CORES * NUM_SUBCORES
ROWS_PER_WORKER = BATCH // NUM_WORKERS
SPLIT = 128
REST = SEQ_LEN - SPLIT
NBUF = 4
NGROUPS = ROWS_PER_WORKER // NBUF
LANES = 16
VPR = DIM // LANES


@jax.jit
def _sc_embed(inputs, token_table, position_table):
  mesh = plsc.VectorSubcoreMesh(
      core_axis_name="c", subcore_axis_name="s",
      num_cores=NUM_CORES, num_subcores=NUM_SUBCORES)

  scratch = (
      [pltpu.VMEM((SEQ_LEN, DIM), jnp.float32)]
      + [pltpu.VMEM((SEQ_LEN,), jnp.int32) for _ in range(NBUF)]
      + [pltpu.VMEM((SEQ_LEN, DIM), jnp.float32) for _ in range(NBUF)]
      + [pltpu.SemaphoreType.DMA for _ in range(3 * NBUF)]
  )

  @functools.partial(
      pl.kernel,
      out_type=jax.ShapeDtypeStruct((BATCH, SEQ_LEN, 2 * DIM), jnp.float32),
      mesh=mesh,
      scratch_types=scratch,
      compiler_params=pltpu.CompilerParams(use_tc_tiling_on_sc=False),
  )
  def k(inputs_hbm, table_hbm, pos_hbm, out_hbm, pos_v, *bufs):
    idx = bufs[:NBUF]
    acc = bufs[NBUF:2 * NBUF]
    si = bufs[2 * NBUF:3 * NBUF]
    sg = bufs[3 * NBUF:4 * NBUF]
    sw = bufs[4 * NBUF:5 * NBUF]

    wid = lax.axis_index("s") * NUM_CORES + lax.axis_index("c")
    base = wid * ROWS_PER_WORKER

    pltpu.sync_copy(pos_hbm, pos_v)

    def start_idx(i, b):
      pltpu.async_copy(inputs_hbm.at[base + i, pl.ds(0, SEQ_LEN)], idx[b],
                       si[b])

    def wait_idx(b):
      pltpu.make_async_copy(inputs_hbm.at[0, pl.ds(0, SEQ_LEN)], idx[b],
                            si[b]).wait()

    def start_gathers(b):
      g0 = pltpu.async_copy(
          table_hbm.at[idx[b].at[pl.ds(0, SPLIT)]],
          acc[b].at[pl.ds(0, SPLIT)], sg[b])
      g1 = pltpu.async_copy(
          table_hbm.at[idx[b].at[pl.ds(SPLIT, REST)]],
          acc[b].at[pl.ds(SPLIT, REST)], sg[b])
      return g0, g1

    def add_pos(b):
      a = acc[b]

      @pl.loop(0, SEQ_LEN, unroll=2)
      def _(r):
        for c in range(VPR):
          sl = pl.ds(c * LANES, LANES)
          plsc.addupdate(a.at[r, sl], pos_v[r, sl])

    def start_write(i, b):
      pltpu.async_copy(
          acc[b], out_hbm.at[base + i, pl.ds(0, SEQ_LEN), pl.ds(0, DIM)],
          sw[b])

    def wait_write(b):
      pltpu.make_async_copy(
          acc[b], out_hbm.at[0, pl.ds(0, SEQ_LEN), pl.ds(0, DIM)],
          sw[b]).wait()

    for b in range(NBUF):
      start_idx(b, b)

    handles = []
    for b in range(NBUF):
      wait_idx(b)
      handles.append(start_gathers(b))
    for b in range(NBUF):
      handles[b][0].wait()
      handles[b][1].wait()
      add_pos(b)
      start_write(b, b)
      start_idx(NBUF + b, b)

    @pl.loop(1, NGROUPS)
    def _(o):
      i0 = o * NBUF
      hs = []
      for b in range(NBUF):
        wait_idx(b)
        wait_write(b)
        hs.append(start_gathers(b))
      for b in range(NBUF):
        hs[b][0].wait()
        hs[b][1].wait()
        add_pos(b)
        start_write(i0 + b, b)

        @pl.when(i0 + b + NBUF < ROWS_PER_WORKER)
        def _():
          start_idx(i0 + b + NBUF, b)

    for b in range(NBUF):
      wait_write(b)

  return k(inputs, token_table, position_table)


def kernel(inputs, token_table, position_table):
  inputs_p = jnp.pad(inputs.astype(jnp.int32), ((0, 0), (0, 56)))
  out = _sc_embed(inputs_p, token_table, position_table)
  return out[:, :, :DIM]
